# Initial kernel scaffold; baseline (speedup 1.0000x reference)
import functools

import jax
import jax.numpy as jnp
from jax import lax
from jax.experimental import pallas as pl
from jax.experimental.pallas import tpu as pltpu
from jax.experimental.pallas import tpu_sc as plsc

_N = 100000
_E = 1600000
_K = 70
_INV_TEMP = 2.0

_NC = 2
_NS = 16
_NW = _NC * _NS
_CH = 128
_NCHUNK = 12512
_EPAD = _NCHUNK * _CH
_CPW = _NCHUNK // _NW
_CPT = _NCHUNK // _NS
_NPAD = 100096
_RPT = _NPAD // _NS
_BLK = 256
_NBLK = _NPAD // _BLK


def _sc_mesh():
    return plsc.VectorSubcoreMesh(
        core_axis_name="c", subcore_axis_name="s",
        num_cores=_NC, num_subcores=_NS)


@functools.partial(
    pl.kernel,
    out_type=jax.ShapeDtypeStruct((_NC * _NPAD, 1), jnp.float32),
    mesh=_sc_mesh(),
    scratch_types=[
        pltpu.VMEM((_CH,), jnp.int32),
        pltpu.VMEM((_CH, 1), jnp.float32),
        pltpu.VMEM_SHARED((_NPAD, 1), jnp.float32),
        pltpu.SemaphoreType.DMA,
    ],
)
def _sc_degree(dst_hbm, ones_hbm, z_hbm, out_hbm, dst_v, ones_v, acc, sem):
    cid = lax.axis_index("c")
    sid = lax.axis_index("s")
    wid = sid * _NC + cid
    r0 = sid * _RPT
    pltpu.sync_copy(z_hbm.at[pl.ds(r0, _RPT)], acc.at[pl.ds(r0, _RPT)])
    pltpu.sync_copy(ones_hbm, ones_v)
    plsc.subcore_barrier()

    def body(j, carry):
        off = (wid * _CPW + j) * _CH
        pltpu.async_copy(dst_hbm.at[pl.ds(off, _CH)], dst_v, sem).wait()
        pltpu.sync_copy(ones_v, acc.at[dst_v], add=True)
        return carry

    lax.fori_loop(0, _CPW, body, 0)
    plsc.subcore_barrier()
    pltpu.sync_copy(acc.at[pl.ds(r0, _RPT)],
                    out_hbm.at[pl.ds(cid * _NPAD + r0, _RPT)])


def _make_sc_agg(d):
    @functools.partial(
        pl.kernel,
        out_type=jax.ShapeDtypeStruct((_NC * _NPAD, d), jnp.float32),
        mesh=_sc_mesh(),
        scratch_types=[
            pltpu.VMEM((_CH,), jnp.int32),
            pltpu.VMEM((_CH,), jnp.int32),
            pltpu.VMEM((_CH, d), jnp.float32),
            pltpu.VMEM_SHARED((_NPAD, d), jnp.float32),
            pltpu.SemaphoreType.DMA,
            pltpu.SemaphoreType.DMA,
            pltpu.SemaphoreType.DMA,
        ],
    )
    def agg(src_hbm, dst_hbm, g_hbm, z_hbm, out_hbm,
            src_v, dst_v, rows_v, acc, s1, s2, s3):
        cid = lax.axis_index("c")
        sid = lax.axis_index("s")
        wid = sid * _NC + cid
        r0 = sid * _RPT
        pltpu.sync_copy(z_hbm.at[pl.ds(r0, _RPT)], acc.at[pl.ds(r0, _RPT)])
        plsc.subcore_barrier()

        def body(j, carry):
            off = (wid * _CPW + j) * _CH
            ca = pltpu.async_copy(src_hbm.at[pl.ds(off, _CH)], src_v, s1)
            cb = pltpu.async_copy(dst_hbm.at[pl.ds(off, _CH)], dst_v, s2)
            ca.wait()
            cb.wait()
            pltpu.async_copy(g_hbm.at[src_v], rows_v, s3).wait()
            pltpu.sync_copy(rows_v, acc.at[dst_v], add=True)
            return carry

        lax.fori_loop(0, _CPW, body, 0)
        plsc.subcore_barrier()
        pltpu.sync_copy(acc.at[pl.ds(r0, _RPT)],
                        out_hbm.at[pl.ds(cid * _NPAD + r0, _RPT)])

    return agg


_sc_agg2 = _make_sc_agg(2)
_sc_agg16 = _make_sc_agg(16)


@functools.partial(
    pl.kernel,
    out_type=jax.ShapeDtypeStruct((_NC * _NPAD, 16), jnp.float32),
    mesh=_sc_mesh(),
    scratch_types=[
        pltpu.VMEM((_CH,), jnp.int32),
        pltpu.VMEM((_CH,), jnp.int32),
        pltpu.VMEM((_CH, 16), jnp.float32),
        pltpu.VMEM_SHARED((_NPAD, 16), jnp.float32),
        pltpu.SemaphoreType.DMA,
        pltpu.SemaphoreType.DMA,
        pltpu.SemaphoreType.DMA,
    ],
)
def _sc_agg_split(src_hbm, dst_hbm, g_hbm, z_hbm, out_hbm,
                  src_v, dst_v, rows_v, acc, s1, s2, s3):
    cid = lax.axis_index("c")
    sid = lax.axis_index("s")
    r0 = sid * _RPT
    tbl_off = cid * _NPAD
    pltpu.sync_copy(z_hbm.at[pl.ds(r0, _RPT)], acc.at[pl.ds(r0, _RPT)])
    plsc.subcore_barrier()

    def body(j, carry):
        off = (sid * _CPT + j) * _CH
        ca = pltpu.async_copy(src_hbm.at[pl.ds(off, _CH)], src_v, s1)
        cb = pltpu.async_copy(dst_hbm.at[pl.ds(off, _CH)], dst_v, s2)
        ca.wait()
        cb.wait()
        for i in range(_CH // 16):
            sl = pl.ds(i * 16, 16)
            src_v[sl] = src_v[sl] + tbl_off
        pltpu.async_copy(g_hbm.at[src_v], rows_v, s3).wait()
        pltpu.sync_copy(rows_v, acc.at[dst_v], add=True)
        return carry

    lax.fori_loop(0, _CPT, body, 0)
    plsc.subcore_barrier()
    pltpu.sync_copy(acc.at[pl.ds(r0, _RPT)],
                    out_hbm.at[pl.ds(cid * _NPAD + r0, _RPT)])


def _tc_prep(cnt2, x_p):
    def body(cnt_ref, x_ref, dinv_ref, g0_ref):
        c = cnt_ref[0] + cnt_ref[1] + 1.0
        dv = lax.rsqrt(c)
        dinv_ref[...] = dv
        g0_ref[...] = x_ref[...] * dv

    return pl.pallas_call(
        body,
        grid=(_NBLK,),
        in_specs=[
            pl.BlockSpec((2, _BLK, 1), lambda i: (0, i, 0)),
            pl.BlockSpec((_BLK, 2), lambda i: (i, 0)),
        ],
        out_specs=[
            pl.BlockSpec((_BLK, 1), lambda i: (i, 0)),
            pl.BlockSpec((_BLK, 2), lambda i: (i, 0)),
        ],
        out_shape=[
            jax.ShapeDtypeStruct((_NPAD, 1), jnp.float32),
            jax.ShapeDtypeStruct((_NPAD, 2), jnp.float32),
        ],
    )(cnt2, x_p)


def _make_tc_layer(d_in, d_out, split_out):
    def body(p_ref, g_ref, dinv_ref, w_ref, b_ref, out_ref):
        dv = dinv_ref[...]
        t = (p_ref[0] + p_ref[1] + g_ref[...]) * dv
        h = jnp.dot(t, w_ref[...], preferred_element_type=jnp.float32)
        h = jnp.maximum(h + b_ref[...], 0.0) * dv
        if split_out:
            out_ref[0] = h[:, :d_out // 2]
            out_ref[1] = h[:, d_out // 2:]
        else:
            out_ref[...] = h

    if split_out:
        out_spec = pl.BlockSpec((2, _BLK, d_out // 2), lambda i: (0, i, 0))
        out_shape = jax.ShapeDtypeStruct((2, _NPAD, d_out // 2), jnp.float32)
    else:
        out_spec = pl.BlockSpec((_BLK, d_out), lambda i: (i, 0))
        out_shape = jax.ShapeDtypeStruct((_NPAD, d_out), jnp.float32)

    return pl.pallas_call(
        body,
        grid=(_NBLK,),
        in_specs=[
            pl.BlockSpec((2, _BLK, d_in), lambda i: (0, i, 0)),
            pl.BlockSpec((_BLK, d_in), lambda i: (i, 0)),
            pl.BlockSpec((_BLK, 1), lambda i: (i, 0)),
            pl.BlockSpec((d_in, d_out), lambda i: (0, 0)),
            pl.BlockSpec((1, d_out), lambda i: (0, 0)),
        ],
        out_specs=out_spec,
        out_shape=out_shape,
    )


def _tc_final(p2, g2, dinv, lg_p, gn_p, W3, b3):
    def body(p_ref, g_ref, dinv_ref, lg_ref, gn_ref, w_ref, b_ref, out_ref):
        dv = dinv_ref[...]
        t = jnp.concatenate([p_ref[0] + g_ref[0], p_ref[1] + g_ref[1]],
                            axis=1) * dv
        h = jnp.dot(t, w_ref[...], preferred_element_type=jnp.float32)
        h = jnp.maximum(h + b_ref[...], 0.0)
        z = (lg_ref[...] + gn_ref[...]) * _INV_TEMP
        z = z - jnp.max(z, axis=1, keepdims=True)
        e = jnp.exp(z)
        y = e / jnp.sum(e, axis=1, keepdims=True)
        row = (pl.program_id(0) * _BLK
               + lax.broadcasted_iota(jnp.int32, (_BLK, 1), 0))
        y = jnp.where(row < _N, y, 0.0)
        contrib = lax.dot_general(
            y, h, (((0,), (0,)), ((), ())),
            preferred_element_type=jnp.float32)

        @pl.when(pl.program_id(0) == 0)
        def _():
            out_ref[...] = jnp.zeros_like(out_ref)

        out_ref[...] += contrib

    return pl.pallas_call(
        body,
        grid=(_NBLK,),
        in_specs=[
            pl.BlockSpec((2, _BLK, 16), lambda i: (0, i, 0)),
            pl.BlockSpec((2, _BLK, 16), lambda i: (0, i, 0)),
            pl.BlockSpec((_BLK, 1), lambda i: (i, 0)),
            pl.BlockSpec((_BLK, _K), lambda i: (i, 0)),
            pl.BlockSpec((_BLK, _K), lambda i: (i, 0)),
            pl.BlockSpec((32, 64), lambda i: (0, 0)),
            pl.BlockSpec((1, 64), lambda i: (0, 0)),
        ],
        out_specs=pl.BlockSpec((_K, 64), lambda i: (0, 0)),
        out_shape=jax.ShapeDtypeStruct((_K, 64), jnp.float32),
    )(p2, g2, dinv, lg_p, gn_p, W3, b3)


_tc_layer1 = _make_tc_layer(2, 16, split_out=False)
_tc_layer2 = _make_tc_layer(16, 32, split_out=True)


def kernel(x, edge_index, logits, gumbel_noise, W1, b1, W2, b2, W3, b3):
    src = edge_index[0]
    dst = edge_index[1]
    pad_e = _EPAD - _E
    src_p = jnp.concatenate([src, jnp.zeros((pad_e,), jnp.int32)])
    dst_p = jnp.concatenate([dst, jnp.full((pad_e,), _N, jnp.int32)])
    x_p = jnp.zeros((_NPAD, 2), jnp.float32).at[:_N].set(x)
    lg_p = jnp.zeros((_NPAD, _K), jnp.float32).at[:_N].set(logits)
    gn_p = jnp.zeros((_NPAD, _K), jnp.float32).at[:_N].set(gumbel_noise)
    z1 = jnp.zeros((_NPAD, 1), jnp.float32)
    z2 = jnp.zeros((_NPAD, 2), jnp.float32)
    z16 = jnp.zeros((_NPAD, 16), jnp.float32)
    ones = jnp.ones((_CH, 1), jnp.float32)

    cnt = _sc_degree(dst_p, ones, z1).reshape(_NC, _NPAD, 1)
    dinv, g0 = _tc_prep(cnt, x_p)
    p0 = _sc_agg2(src_p, dst_p, g0, z2).reshape(_NC, _NPAD, 2)
    g1 = _tc_layer1(p0, g0, dinv, W1, b1.reshape(1, 16))
    p1 = _sc_agg16(src_p, dst_p, g1, z16).reshape(_NC, _NPAD, 16)
    g2 = _tc_layer2(p1, g1, dinv, W2, b2.reshape(1, 32))
    g2s = g2.reshape(_NC * _NPAD, 16)
    p2 = _sc_agg_split(src_p, dst_p, g2s, z16).reshape(_NC, _NPAD, 16)
    out = _tc_final(p2, g2, dinv, lg_p, gn_p, W3, b3.reshape(1, 64))
    return out.reshape(1, -1)


# speedup vs baseline: 12.1740x; 12.1740x over previous
import functools

import jax
import jax.numpy as jnp
from jax import lax
from jax.experimental import pallas as pl
from jax.experimental.pallas import tpu as pltpu
from jax.experimental.pallas import tpu_sc as plsc

_N = 100000
_E = 1600000
_K = 70
_INV_TEMP = 2.0

_NC = 2
_NS = 16
_NW = _NC * _NS
_CH = 128
_NCHUNK = 12512
_EPAD = _NCHUNK * _CH
_CPW = _NCHUNK // _NW
_CPT = _NCHUNK // _NS
_NPAD = 100096
_RPT = _NPAD // _NS
_BLK = 256
_NBLK = _NPAD // _BLK


def _sc_mesh():
    return plsc.VectorSubcoreMesh(
        core_axis_name="c", subcore_axis_name="s",
        num_cores=_NC, num_subcores=_NS)


@functools.partial(
    pl.kernel,
    out_type=jax.ShapeDtypeStruct((_NC * _NPAD, 8), jnp.float32),
    mesh=_sc_mesh(),
    compiler_params=pltpu.CompilerParams(use_tc_tiling_on_sc=False),
    scratch_types=[
        pltpu.VMEM((_CH,), jnp.int32),
        pltpu.VMEM((_CH, 8), jnp.float32),
        pltpu.VMEM_SHARED((_NPAD, 8), jnp.float32),
        pltpu.SemaphoreType.DMA,
    ],
)
def _sc_degree(dst_hbm, ones_hbm, z_hbm, out_hbm, dst_v, ones_v, acc, sem):
    cid = lax.axis_index("c")
    sid = lax.axis_index("s")
    wid = sid * _NC + cid
    r0 = sid * _RPT
    pltpu.sync_copy(z_hbm.at[pl.ds(r0, _RPT)], acc.at[pl.ds(r0, _RPT)])
    pltpu.sync_copy(ones_hbm, ones_v)
    plsc.subcore_barrier()

    def body(j, carry):
        off = (wid * _CPW + j) * _CH
        pltpu.async_copy(dst_hbm.at[pl.ds(off, _CH)], dst_v, sem).wait()
        pltpu.sync_copy(ones_v, acc.at[dst_v], add=True)
        return carry

    lax.fori_loop(0, _CPW, body, 0)
    plsc.subcore_barrier()
    pltpu.sync_copy(acc.at[pl.ds(r0, _RPT)],
                    out_hbm.at[pl.ds(cid * _NPAD + r0, _RPT)])


def _make_sc_agg(d):
    @functools.partial(
        pl.kernel,
        out_type=jax.ShapeDtypeStruct((_NC * _NPAD, d), jnp.float32),
        mesh=_sc_mesh(),
        compiler_params=pltpu.CompilerParams(use_tc_tiling_on_sc=False),
        scratch_types=[
            pltpu.VMEM((_CH,), jnp.int32),
            pltpu.VMEM((_CH,), jnp.int32),
            pltpu.VMEM((_CH, d), jnp.float32),
            pltpu.VMEM_SHARED((_NPAD, d), jnp.float32),
            pltpu.SemaphoreType.DMA,
            pltpu.SemaphoreType.DMA,
            pltpu.SemaphoreType.DMA,
        ],
    )
    def agg(src_hbm, dst_hbm, g_hbm, z_hbm, out_hbm,
            src_v, dst_v, rows_v, acc, s1, s2, s3):
        cid = lax.axis_index("c")
        sid = lax.axis_index("s")
        wid = sid * _NC + cid
        r0 = sid * _RPT
        pltpu.sync_copy(z_hbm.at[pl.ds(r0, _RPT)], acc.at[pl.ds(r0, _RPT)])
        plsc.subcore_barrier()

        def body(j, carry):
            off = (wid * _CPW + j) * _CH
            ca = pltpu.async_copy(src_hbm.at[pl.ds(off, _CH)], src_v, s1)
            cb = pltpu.async_copy(dst_hbm.at[pl.ds(off, _CH)], dst_v, s2)
            ca.wait()
            cb.wait()
            pltpu.async_copy(g_hbm.at[src_v], rows_v, s3).wait()
            pltpu.sync_copy(rows_v, acc.at[dst_v], add=True)
            return carry

        lax.fori_loop(0, _CPW, body, 0)
        plsc.subcore_barrier()
        pltpu.sync_copy(acc.at[pl.ds(r0, _RPT)],
                        out_hbm.at[pl.ds(cid * _NPAD + r0, _RPT)])

    return agg


_sc_agg8 = _make_sc_agg(8)
_sc_agg16 = _make_sc_agg(16)


@functools.partial(
    pl.kernel,
    out_type=jax.ShapeDtypeStruct((_NC * _NPAD, 16), jnp.float32),
    mesh=_sc_mesh(),
    compiler_params=pltpu.CompilerParams(use_tc_tiling_on_sc=False),
    scratch_types=[
        pltpu.VMEM((_CH,), jnp.int32),
        pltpu.VMEM((_CH,), jnp.int32),
        pltpu.VMEM((_CH, 16), jnp.float32),
        pltpu.VMEM_SHARED((_NPAD, 16), jnp.float32),
        pltpu.SemaphoreType.DMA,
        pltpu.SemaphoreType.DMA,
        pltpu.SemaphoreType.DMA,
    ],
)
def _sc_agg_split(src_hbm, dst_hbm, g_hbm, z_hbm, out_hbm,
                  src_v, dst_v, rows_v, acc, s1, s2, s3):
    cid = lax.axis_index("c")
    sid = lax.axis_index("s")
    r0 = sid * _RPT
    tbl_off = cid * _NPAD
    pltpu.sync_copy(z_hbm.at[pl.ds(r0, _RPT)], acc.at[pl.ds(r0, _RPT)])
    plsc.subcore_barrier()

    def body(j, carry):
        off = (sid * _CPT + j) * _CH
        ca = pltpu.async_copy(src_hbm.at[pl.ds(off, _CH)], src_v, s1)
        cb = pltpu.async_copy(dst_hbm.at[pl.ds(off, _CH)], dst_v, s2)
        ca.wait()
        cb.wait()
        for i in range(_CH // 16):
            sl = pl.ds(i * 16, 16)
            src_v[sl] = src_v[sl] + tbl_off
        pltpu.async_copy(g_hbm.at[src_v], rows_v, s3).wait()
        pltpu.sync_copy(rows_v, acc.at[dst_v], add=True)
        return carry

    lax.fori_loop(0, _CPT, body, 0)
    plsc.subcore_barrier()
    pltpu.sync_copy(acc.at[pl.ds(r0, _RPT)],
                    out_hbm.at[pl.ds(cid * _NPAD + r0, _RPT)])


def _tc_prep(cnt2, x_p):
    def body(cnt_ref, x_ref, dinv_ref, g0_ref):
        c = cnt_ref[0, :, 0:1] + cnt_ref[1, :, 0:1] + 1.0
        dv = lax.rsqrt(c)
        dinv_ref[...] = dv
        g0_ref[...] = jnp.concatenate(
            [x_ref[...] * dv, jnp.zeros((_BLK, 6), jnp.float32)], axis=1)

    return pl.pallas_call(
        body,
        grid=(_NBLK,),
        in_specs=[
            pl.BlockSpec((2, _BLK, 8), lambda i: (0, i, 0)),
            pl.BlockSpec((_BLK, 2), lambda i: (i, 0)),
        ],
        out_specs=[
            pl.BlockSpec((_BLK, 1), lambda i: (i, 0)),
            pl.BlockSpec((_BLK, 8), lambda i: (i, 0)),
        ],
        out_shape=[
            jax.ShapeDtypeStruct((_NPAD, 1), jnp.float32),
            jax.ShapeDtypeStruct((_NPAD, 8), jnp.float32),
        ],
    )(cnt2, x_p)


def _make_tc_layer(d_in, d_out, split_out, d_mat=None):
    d_mat = d_mat or d_in

    def body(p_ref, g_ref, dinv_ref, w_ref, b_ref, out_ref):
        dv = dinv_ref[...]
        t = (p_ref[0] + p_ref[1] + g_ref[...]) * dv
        t = t[:, :d_mat]
        h = jnp.dot(t, w_ref[...], preferred_element_type=jnp.float32)
        h = jnp.maximum(h + b_ref[...], 0.0) * dv
        if split_out:
            out_ref[0] = h[:, :d_out // 2]
            out_ref[1] = h[:, d_out // 2:]
        else:
            out_ref[...] = h

    if split_out:
        out_spec = pl.BlockSpec((2, _BLK, d_out // 2), lambda i: (0, i, 0))
        out_shape = jax.ShapeDtypeStruct((2, _NPAD, d_out // 2), jnp.float32)
    else:
        out_spec = pl.BlockSpec((_BLK, d_out), lambda i: (i, 0))
        out_shape = jax.ShapeDtypeStruct((_NPAD, d_out), jnp.float32)

    return pl.pallas_call(
        body,
        grid=(_NBLK,),
        in_specs=[
            pl.BlockSpec((2, _BLK, d_in), lambda i: (0, i, 0)),
            pl.BlockSpec((_BLK, d_in), lambda i: (i, 0)),
            pl.BlockSpec((_BLK, 1), lambda i: (i, 0)),
            pl.BlockSpec((d_mat, d_out), lambda i: (0, 0)),
            pl.BlockSpec((1, d_out), lambda i: (0, 0)),
        ],
        out_specs=out_spec,
        out_shape=out_shape,
    )


def _tc_final(p2, g2, dinv, lg_p, gn_p, W3, b3):
    def body(p_ref, g_ref, dinv_ref, lg_ref, gn_ref, w_ref, b_ref, out_ref):
        dv = dinv_ref[...]
        t = jnp.concatenate([p_ref[0] + g_ref[0], p_ref[1] + g_ref[1]],
                            axis=1) * dv
        h = jnp.dot(t, w_ref[...], preferred_element_type=jnp.float32)
        h = jnp.maximum(h + b_ref[...], 0.0)
        z = (lg_ref[...] + gn_ref[...]) * _INV_TEMP
        z = z - jnp.max(z, axis=1, keepdims=True)
        e = jnp.exp(z)
        y = e / jnp.sum(e, axis=1, keepdims=True)
        row = (pl.program_id(0) * _BLK
               + lax.broadcasted_iota(jnp.int32, (_BLK, 1), 0))
        y = jnp.where(row < _N, y, 0.0)
        contrib = lax.dot_general(
            y, h, (((0,), (0,)), ((), ())),
            preferred_element_type=jnp.float32)

        @pl.when(pl.program_id(0) == 0)
        def _():
            out_ref[...] = jnp.zeros_like(out_ref)

        out_ref[...] += contrib

    return pl.pallas_call(
        body,
        grid=(_NBLK,),
        in_specs=[
            pl.BlockSpec((2, _BLK, 16), lambda i: (0, i, 0)),
            pl.BlockSpec((2, _BLK, 16), lambda i: (0, i, 0)),
            pl.BlockSpec((_BLK, 1), lambda i: (i, 0)),
            pl.BlockSpec((_BLK, _K), lambda i: (i, 0)),
            pl.BlockSpec((_BLK, _K), lambda i: (i, 0)),
            pl.BlockSpec((32, 64), lambda i: (0, 0)),
            pl.BlockSpec((1, 64), lambda i: (0, 0)),
        ],
        out_specs=pl.BlockSpec((_K, 64), lambda i: (0, 0)),
        out_shape=jax.ShapeDtypeStruct((_K, 64), jnp.float32),
    )(p2, g2, dinv, lg_p, gn_p, W3, b3)


_tc_layer1 = _make_tc_layer(8, 16, split_out=False, d_mat=2)
_tc_layer2 = _make_tc_layer(16, 32, split_out=True)


def kernel(x, edge_index, logits, gumbel_noise, W1, b1, W2, b2, W3, b3):
    src = edge_index[0]
    dst = edge_index[1]
    pad_e = _EPAD - _E
    src_p = jnp.concatenate([src, jnp.zeros((pad_e,), jnp.int32)])
    dst_p = jnp.concatenate([dst, jnp.full((pad_e,), _N, jnp.int32)])
    x_p = jnp.zeros((_NPAD, 2), jnp.float32).at[:_N].set(x)
    lg_p = jnp.zeros((_NPAD, _K), jnp.float32).at[:_N].set(logits)
    gn_p = jnp.zeros((_NPAD, _K), jnp.float32).at[:_N].set(gumbel_noise)
    z8 = jnp.zeros((_NPAD, 8), jnp.float32)
    z16 = jnp.zeros((_NPAD, 16), jnp.float32)
    ones = jnp.ones((_CH, 8), jnp.float32)

    cnt = _sc_degree(dst_p, ones, z8).reshape(_NC, _NPAD, 8)
    dinv, g0 = _tc_prep(cnt, x_p)
    p0 = _sc_agg8(src_p, dst_p, g0, z8).reshape(_NC, _NPAD, 8)
    g1 = _tc_layer1(p0, g0, dinv, W1, b1.reshape(1, 16))
    p1 = _sc_agg16(src_p, dst_p, g1, z16).reshape(_NC, _NPAD, 16)
    g2 = _tc_layer2(p1, g1, dinv, W2, b2.reshape(1, 32))
    g2s = g2.reshape(_NC * _NPAD, 16)
    p2 = _sc_agg_split(src_p, dst_p, g2s, z16).reshape(_NC, _NPAD, 16)
    out = _tc_final(p2, g2, dinv, lg_p, gn_p, W3, b3.reshape(1, 64))
    return out.reshape(1, -1)


# speedup vs baseline: 21.6505x; 1.7784x over previous
import functools

import jax
import jax.numpy as jnp
from jax import lax
from jax.experimental import pallas as pl
from jax.experimental.pallas import tpu as pltpu
from jax.experimental.pallas import tpu_sc as plsc

_N = 100000
_E = 1600000
_K = 70
_INV_TEMP = 2.0

_NC = 2
_NS = 16
_NW = _NC * _NS
_CH = 128
_G = 8
_NCHUNK = 12544
_EPAD = _NCHUNK * _CH
_CPW = _NCHUNK // _NW
_CPT = _NCHUNK // _NS
_SB = 56
_NGB = _SB // _G
_NPAD = 100096
_RPT = _NPAD // _NS
_BLK = 256
_NBLK = _NPAD // _BLK


def _sc_mesh():
    return plsc.VectorSubcoreMesh(
        core_axis_name="c", subcore_axis_name="s",
        num_cores=_NC, num_subcores=_NS)


_SC_PARAMS = dict(
    mesh=_sc_mesh(),
    compiler_params=pltpu.CompilerParams(use_tc_tiling_on_sc=False),
)


@functools.partial(
    pl.kernel,
    out_type=jax.ShapeDtypeStruct((_NC * _NPAD, 8), jnp.float32),
    scratch_types=[
        pltpu.VMEM((_SB, _CH), jnp.int32),
        pltpu.VMEM((_CH, 8), jnp.float32),
        pltpu.VMEM_SHARED((_NPAD, 8), jnp.float32),
        pltpu.SemaphoreType.DMA,
        pltpu.SemaphoreType.DMA,
    ],
    **_SC_PARAMS,
)
def _sc_degree(dst_hbm, ones_hbm, z_hbm, out_hbm, dst_all, ones_v, acc,
               si, ss):
    cid = lax.axis_index("c")
    sid = lax.axis_index("s")
    wid = sid * _NC + cid
    r0 = sid * _RPT
    pltpu.sync_copy(z_hbm.at[pl.ds(r0, _RPT)], acc.at[pl.ds(r0, _RPT)])
    pltpu.sync_copy(ones_hbm, ones_v)
    plsc.subcore_barrier()

    def block(bi, carry):
        c0 = wid * _CPW + bi * _SB
        pltpu.async_copy(dst_hbm.at[pl.ds(c0, _SB)], dst_all, si).wait()

        def group(g, carry2):
            k0 = g * _G
            cs = [pltpu.async_copy(ones_v, acc.at[dst_all.at[k0 + b]], ss,
                                   add=True)
                  for b in range(_G)]
            for c in cs:
                c.wait()
            return carry2

        lax.fori_loop(0, _NGB, group, 0)
        return carry

    lax.fori_loop(0, _CPW // _SB, block, 0)
    plsc.subcore_barrier()
    pltpu.sync_copy(acc.at[pl.ds(r0, _RPT)],
                    out_hbm.at[pl.ds(cid * _NPAD + r0, _RPT)])


def _make_sc_agg(d):
    @functools.partial(
        pl.kernel,
        out_type=jax.ShapeDtypeStruct((_NC * _NPAD, d), jnp.float32),
        scratch_types=[
            pltpu.VMEM((_SB, _CH), jnp.int32),
            pltpu.VMEM((_SB, _CH), jnp.int32),
            pltpu.VMEM((_G, _CH, d), jnp.float32),
            pltpu.VMEM_SHARED((_NPAD, d), jnp.float32),
            pltpu.SemaphoreType.DMA,
            pltpu.SemaphoreType.DMA,
            pltpu.SemaphoreType.DMA,
        ],
        **_SC_PARAMS,
    )
    def agg(src_hbm, dst_hbm, g_hbm, z_hbm, out_hbm,
            src_all, dst_all, rows, acc, si, sg, ss):
        cid = lax.axis_index("c")
        sid = lax.axis_index("s")
        wid = sid * _NC + cid
        r0 = sid * _RPT
        pltpu.sync_copy(z_hbm.at[pl.ds(r0, _RPT)], acc.at[pl.ds(r0, _RPT)])
        plsc.subcore_barrier()

        def block(bi, carry):
            c0 = wid * _CPW + bi * _SB
            ca = pltpu.async_copy(src_hbm.at[pl.ds(c0, _SB)], src_all, si)
            cb = pltpu.async_copy(dst_hbm.at[pl.ds(c0, _SB)], dst_all, si)
            ca.wait()
            cb.wait()

            def group(g, carry2):
                k0 = g * _G
                gs = [pltpu.async_copy(g_hbm.at[src_all.at[k0 + b]],
                                       rows.at[b], sg)
                      for b in range(_G)]
                cs = []
                for b in range(_G):
                    gs[b].wait()
                    cs.append(pltpu.async_copy(rows.at[b],
                                               acc.at[dst_all.at[k0 + b]],
                                               ss, add=True))
                for c in cs:
                    c.wait()
                return carry2

            lax.fori_loop(0, _NGB, group, 0)
            return carry

        lax.fori_loop(0, _CPW // _SB, block, 0)
        plsc.subcore_barrier()
        pltpu.sync_copy(acc.at[pl.ds(r0, _RPT)],
                        out_hbm.at[pl.ds(cid * _NPAD + r0, _RPT)])

    return agg


_sc_agg8 = _make_sc_agg(8)
_sc_agg16 = _make_sc_agg(16)


@functools.partial(
    pl.kernel,
    out_type=jax.ShapeDtypeStruct((_NC * _NPAD, 16), jnp.float32),
    scratch_types=[
        pltpu.VMEM((_SB, _CH), jnp.int32),
        pltpu.VMEM((_SB, _CH), jnp.int32),
        pltpu.VMEM((_G, _CH, 16), jnp.float32),
        pltpu.VMEM_SHARED((_NPAD, 16), jnp.float32),
        pltpu.SemaphoreType.DMA,
        pltpu.SemaphoreType.DMA,
        pltpu.SemaphoreType.DMA,
    ],
    **_SC_PARAMS,
)
def _sc_agg_split(src_hbm, srcoff_hbm, dst_hbm, g_hbm, z_hbm, out_hbm,
                  src_all, dst_all, rows, acc, si, sg, ss):
    cid = lax.axis_index("c")
    sid = lax.axis_index("s")
    r0 = sid * _RPT
    pltpu.sync_copy(z_hbm.at[pl.ds(r0, _RPT)], acc.at[pl.ds(r0, _RPT)])
    plsc.subcore_barrier()

    def block(bi, carry):
        c0 = sid * _CPT + bi * _SB

        @pl.when(cid == 0)
        def _():
            pltpu.sync_copy(src_hbm.at[pl.ds(c0, _SB)], src_all)

        @pl.when(cid == 1)
        def _():
            pltpu.sync_copy(srcoff_hbm.at[pl.ds(c0, _SB)], src_all)

        pltpu.sync_copy(dst_hbm.at[pl.ds(c0, _SB)], dst_all)

        def group(g, carry2):
            k0 = g * _G
            gs = [pltpu.async_copy(g_hbm.at[src_all.at[k0 + b]], rows.at[b],
                                   sg)
                  for b in range(_G)]
            cs = []
            for b in range(_G):
                gs[b].wait()
                cs.append(pltpu.async_copy(rows.at[b],
                                           acc.at[dst_all.at[k0 + b]], ss,
                                           add=True))
            for c in cs:
                c.wait()
            return carry2

        lax.fori_loop(0, _NGB, group, 0)
        return carry

    lax.fori_loop(0, _CPT // _SB, block, 0)
    plsc.subcore_barrier()
    pltpu.sync_copy(acc.at[pl.ds(r0, _RPT)],
                    out_hbm.at[pl.ds(cid * _NPAD + r0, _RPT)])


def _tc_prep(cnt2, x_p):
    def body(cnt_ref, x_ref, dinv_ref, g0_ref):
        c = cnt_ref[0, :, 0:1] + cnt_ref[1, :, 0:1] + 1.0
        dv = lax.rsqrt(c)
        dinv_ref[...] = dv
        g0_ref[...] = jnp.concatenate(
            [x_ref[...] * dv, jnp.zeros((_BLK, 6), jnp.float32)], axis=1)

    return pl.pallas_call(
        body,
        grid=(_NBLK,),
        in_specs=[
            pl.BlockSpec((2, _BLK, 8), lambda i: (0, i, 0)),
            pl.BlockSpec((_BLK, 2), lambda i: (i, 0)),
        ],
        out_specs=[
            pl.BlockSpec((_BLK, 1), lambda i: (i, 0)),
            pl.BlockSpec((_BLK, 8), lambda i: (i, 0)),
        ],
        out_shape=[
            jax.ShapeDtypeStruct((_NPAD, 1), jnp.float32),
            jax.ShapeDtypeStruct((_NPAD, 8), jnp.float32),
        ],
    )(cnt2, x_p)


def _make_tc_layer(d_in, d_out, split_out, d_mat=None):
    d_mat = d_mat or d_in

    def body(p_ref, g_ref, dinv_ref, w_ref, b_ref, out_ref):
        dv = dinv_ref[...]
        t = (p_ref[0] + p_ref[1] + g_ref[...]) * dv
        t = t[:, :d_mat]
        h = jnp.dot(t, w_ref[...], preferred_element_type=jnp.float32)
        h = jnp.maximum(h + b_ref[...], 0.0) * dv
        if split_out:
            out_ref[0] = h[:, :d_out // 2]
            out_ref[1] = h[:, d_out // 2:]
        else:
            out_ref[...] = h

    if split_out:
        out_spec = pl.BlockSpec((2, _BLK, d_out // 2), lambda i: (0, i, 0))
        out_shape = jax.ShapeDtypeStruct((2, _NPAD, d_out // 2), jnp.float32)
    else:
        out_spec = pl.BlockSpec((_BLK, d_out), lambda i: (i, 0))
        out_shape = jax.ShapeDtypeStruct((_NPAD, d_out), jnp.float32)

    return pl.pallas_call(
        body,
        grid=(_NBLK,),
        in_specs=[
            pl.BlockSpec((2, _BLK, d_in), lambda i: (0, i, 0)),
            pl.BlockSpec((_BLK, d_in), lambda i: (i, 0)),
            pl.BlockSpec((_BLK, 1), lambda i: (i, 0)),
            pl.BlockSpec((d_mat, d_out), lambda i: (0, 0)),
            pl.BlockSpec((1, d_out), lambda i: (0, 0)),
        ],
        out_specs=out_spec,
        out_shape=out_shape,
    )


def _tc_final(p2, g2, dinv, lg_p, gn_p, W3, b3):
    def body(p_ref, g_ref, dinv_ref, lg_ref, gn_ref, w_ref, b_ref, out_ref):
        dv = dinv_ref[...]
        t = jnp.concatenate([p_ref[0] + g_ref[0], p_ref[1] + g_ref[1]],
                            axis=1) * dv
        h = jnp.dot(t, w_ref[...], preferred_element_type=jnp.float32)
        h = jnp.maximum(h + b_ref[...], 0.0)
        z = (lg_ref[...] + gn_ref[...]) * _INV_TEMP
        z = z - jnp.max(z, axis=1, keepdims=True)
        e = jnp.exp(z)
        y = e / jnp.sum(e, axis=1, keepdims=True)
        row = (pl.program_id(0) * _BLK
               + lax.broadcasted_iota(jnp.int32, (_BLK, 1), 0))
        y = jnp.where(row < _N, y, 0.0)
        contrib = lax.dot_general(
            y, h, (((0,), (0,)), ((), ())),
            preferred_element_type=jnp.float32)

        @pl.when(pl.program_id(0) == 0)
        def _():
            out_ref[...] = jnp.zeros_like(out_ref)

        out_ref[...] += contrib

    return pl.pallas_call(
        body,
        grid=(_NBLK,),
        in_specs=[
            pl.BlockSpec((2, _BLK, 16), lambda i: (0, i, 0)),
            pl.BlockSpec((2, _BLK, 16), lambda i: (0, i, 0)),
            pl.BlockSpec((_BLK, 1), lambda i: (i, 0)),
            pl.BlockSpec((_BLK, _K), lambda i: (i, 0)),
            pl.BlockSpec((_BLK, _K), lambda i: (i, 0)),
            pl.BlockSpec((32, 64), lambda i: (0, 0)),
            pl.BlockSpec((1, 64), lambda i: (0, 0)),
        ],
        out_specs=pl.BlockSpec((_K, 64), lambda i: (0, 0)),
        out_shape=jax.ShapeDtypeStruct((_K, 64), jnp.float32),
    )(p2, g2, dinv, lg_p, gn_p, W3, b3)


_tc_layer1 = _make_tc_layer(8, 16, split_out=False, d_mat=2)
_tc_layer2 = _make_tc_layer(16, 32, split_out=True)


def kernel(x, edge_index, logits, gumbel_noise, W1, b1, W2, b2, W3, b3):
    src = edge_index[0]
    dst = edge_index[1]
    pad_e = _EPAD - _E
    src_p = jnp.concatenate([src, jnp.zeros((pad_e,), jnp.int32)]
                            ).reshape(_NCHUNK, _CH)
    dst_p = jnp.concatenate([dst, jnp.full((pad_e,), _N, jnp.int32)]
                            ).reshape(_NCHUNK, _CH)
    srcoff_p = src_p + _NPAD
    x_p = jnp.zeros((_NPAD, 2), jnp.float32).at[:_N].set(x)
    lg_p = jnp.zeros((_NPAD, _K), jnp.float32).at[:_N].set(logits)
    gn_p = jnp.zeros((_NPAD, _K), jnp.float32).at[:_N].set(gumbel_noise)
    z8 = jnp.zeros((_NPAD, 8), jnp.float32)
    z16 = jnp.zeros((_NPAD, 16), jnp.float32)
    ones = jnp.ones((_CH, 8), jnp.float32)

    cnt = _sc_degree(dst_p, ones, z8).reshape(_NC, _NPAD, 8)
    dinv, g0 = _tc_prep(cnt, x_p)
    p0 = _sc_agg8(src_p, dst_p, g0, z8).reshape(_NC, _NPAD, 8)
    g1 = _tc_layer1(p0, g0, dinv, W1, b1.reshape(1, 16))
    p1 = _sc_agg16(src_p, dst_p, g1, z16).reshape(_NC, _NPAD, 16)
    g2 = _tc_layer2(p1, g1, dinv, W2, b2.reshape(1, 32))
    g2s = g2.reshape(_NC * _NPAD, 16)
    p2 = _sc_agg_split(src_p, srcoff_p, dst_p, g2s, z16).reshape(
        _NC, _NPAD, 16)
    out = _tc_final(p2, g2, dinv, lg_p, gn_p, W3, b3.reshape(1, 64))
    return out.reshape(1, -1)


# speedup vs baseline: 44.8771x; 2.0728x over previous
import functools

import jax
import jax.numpy as jnp
from jax import lax
from jax.experimental import pallas as pl
from jax.experimental.pallas import tpu as pltpu
from jax.experimental.pallas import tpu_sc as plsc

_N = 100000
_E = 1600000
_K = 70
_INV_TEMP = 2.0
_F = 16

_NC = 2
_NS = 16
_NW = _NC * _NS
_CH = 128
_G = 8
_NCHUNK = 12544
_EPAD = _NCHUNK * _CH
_CPW = _NCHUNK // _NW
_CPT = _NCHUNK // _NS
_SB = 56
_NGB = _SB // _G
_NPAD = 100096
_RPT = _NPAD // _NS
_NFR = _NPAD * _F // 128
_NBLK = 17
_BF = _NFR // _NBLK
_BLK = _NPAD // _NBLK


def _sc_mesh():
    return plsc.VectorSubcoreMesh(
        core_axis_name="c", subcore_axis_name="s",
        num_cores=_NC, num_subcores=_NS)


_SC_PARAMS = dict(
    mesh=_sc_mesh(),
    compiler_params=pltpu.CompilerParams(use_tc_tiling_on_sc=False),
)


@functools.partial(
    pl.kernel,
    out_type=jax.ShapeDtypeStruct((_NC * _NPAD, _F), jnp.float32),
    scratch_types=[
        pltpu.VMEM((_SB, _CH), jnp.int32),
        pltpu.VMEM((_CH, _F), jnp.float32),
        pltpu.VMEM_SHARED((_NPAD, _F), jnp.float32),
        pltpu.SemaphoreType.DMA,
        pltpu.SemaphoreType.DMA,
    ],
    **_SC_PARAMS,
)
def _sc_degree(dst_hbm, ones_hbm, z_hbm, out_hbm, dst_all, ones_v, acc,
               si, ss):
    cid = lax.axis_index("c")
    sid = lax.axis_index("s")
    wid = sid * _NC + cid
    r0 = sid * _RPT
    pltpu.sync_copy(z_hbm.at[pl.ds(r0, _RPT)], acc.at[pl.ds(r0, _RPT)])
    pltpu.sync_copy(ones_hbm, ones_v)
    plsc.subcore_barrier()

    def block(bi, carry):
        c0 = wid * _CPW + bi * _SB
        pltpu.async_copy(dst_hbm.at[pl.ds(c0, _SB)], dst_all, si).wait()

        def group(g, carry2):
            k0 = g * _G
            cs = [pltpu.async_copy(ones_v, acc.at[dst_all.at[k0 + b]], ss,
                                   add=True)
                  for b in range(_G)]
            for c in cs:
                c.wait()
            return carry2

        lax.fori_loop(0, _NGB, group, 0)
        return carry

    lax.fori_loop(0, _CPW // _SB, block, 0)
    plsc.subcore_barrier()
    pltpu.sync_copy(acc.at[pl.ds(r0, _RPT)],
                    out_hbm.at[pl.ds(cid * _NPAD + r0, _RPT)])


@functools.partial(
    pl.kernel,
    out_type=jax.ShapeDtypeStruct((_NC * _NPAD, _F), jnp.float32),
    scratch_types=[
        pltpu.VMEM((_SB, _CH), jnp.int32),
        pltpu.VMEM((_SB, _CH), jnp.int32),
        pltpu.VMEM((_G, _CH, _F), jnp.float32),
        pltpu.VMEM_SHARED((_NPAD, _F), jnp.float32),
        pltpu.SemaphoreType.DMA,
        pltpu.SemaphoreType.DMA,
        pltpu.SemaphoreType.DMA,
    ],
    **_SC_PARAMS,
)
def _sc_agg(src_hbm, dst_hbm, g_hbm, z_hbm, out_hbm,
            src_all, dst_all, rows, acc, si, sg, ss):
    cid = lax.axis_index("c")
    sid = lax.axis_index("s")
    wid = sid * _NC + cid
    r0 = sid * _RPT
    pltpu.sync_copy(z_hbm.at[pl.ds(r0, _RPT)], acc.at[pl.ds(r0, _RPT)])
    plsc.subcore_barrier()

    def block(bi, carry):
        c0 = wid * _CPW + bi * _SB
        ca = pltpu.async_copy(src_hbm.at[pl.ds(c0, _SB)], src_all, si)
        cb = pltpu.async_copy(dst_hbm.at[pl.ds(c0, _SB)], dst_all, si)
        ca.wait()
        cb.wait()

        def group(g, carry2):
            k0 = g * _G
            gs = [pltpu.async_copy(g_hbm.at[src_all.at[k0 + b]],
                                   rows.at[b], sg)
                  for b in range(_G)]
            cs = []
            for b in range(_G):
                gs[b].wait()
                cs.append(pltpu.async_copy(rows.at[b],
                                           acc.at[dst_all.at[k0 + b]],
                                           ss, add=True))
            for c in cs:
                c.wait()
            return carry2

        lax.fori_loop(0, _NGB, group, 0)
        return carry

    lax.fori_loop(0, _CPW // _SB, block, 0)
    plsc.subcore_barrier()
    pltpu.sync_copy(acc.at[pl.ds(r0, _RPT)],
                    out_hbm.at[pl.ds(cid * _NPAD + r0, _RPT)])


@functools.partial(
    pl.kernel,
    out_type=jax.ShapeDtypeStruct((_NC * _NPAD, _F), jnp.float32),
    scratch_types=[
        pltpu.VMEM((_SB, _CH), jnp.int32),
        pltpu.VMEM((_SB, _CH), jnp.int32),
        pltpu.VMEM((_G, _CH, _F), jnp.float32),
        pltpu.VMEM_SHARED((_NPAD, _F), jnp.float32),
        pltpu.SemaphoreType.DMA,
        pltpu.SemaphoreType.DMA,
        pltpu.SemaphoreType.DMA,
    ],
    **_SC_PARAMS,
)
def _sc_agg_split(src_hbm, dst_hbm, ga_hbm, gb_hbm, z_hbm, out_hbm,
                  src_all, dst_all, rows, acc, si, sg, ss):
    cid = lax.axis_index("c")
    sid = lax.axis_index("s")
    r0 = sid * _RPT
    pltpu.sync_copy(z_hbm.at[pl.ds(r0, _RPT)], acc.at[pl.ds(r0, _RPT)])
    plsc.subcore_barrier()

    def block(bi, carry):
        c0 = sid * _CPT + bi * _SB
        ca = pltpu.async_copy(src_hbm.at[pl.ds(c0, _SB)], src_all, si)
        cb = pltpu.async_copy(dst_hbm.at[pl.ds(c0, _SB)], dst_all, si)
        ca.wait()
        cb.wait()

        def run(tbl):
            def group(g, carry2):
                k0 = g * _G
                gs = [pltpu.async_copy(tbl.at[src_all.at[k0 + b]],
                                       rows.at[b], sg)
                      for b in range(_G)]
                cs = []
                for b in range(_G):
                    gs[b].wait()
                    cs.append(pltpu.async_copy(rows.at[b],
                                               acc.at[dst_all.at[k0 + b]],
                                               ss, add=True))
                for c in cs:
                    c.wait()
                return carry2

            lax.fori_loop(0, _NGB, group, 0)

        @pl.when(cid == 0)
        def _():
            run(ga_hbm)

        @pl.when(cid == 1)
        def _():
            run(gb_hbm)

        return carry

    lax.fori_loop(0, _CPT // _SB, block, 0)
    plsc.subcore_barrier()
    pltpu.sync_copy(acc.at[pl.ds(r0, _RPT)],
                    out_hbm.at[pl.ds(cid * _NPAD + r0, _RPT)])


_FLAT = pl.BlockSpec((_BF, 128), lambda i: (i, 0))
_FLAT_HI = pl.BlockSpec((_BF, 128), lambda i: (_NBLK + i, 0))
_W128 = pl.BlockSpec((128, 128), lambda i: (0, 0))
_B128 = pl.BlockSpec((1, 128), lambda i: (0, 0))


def _tc_prep(cnt2, x16):
    def body(ca_ref, cb_ref, x_ref, dinv_ref, g0_ref):
        dv = lax.rsqrt(ca_ref[...] + cb_ref[...] + 1.0)
        dinv_ref[...] = dv
        g0_ref[...] = x_ref[...] * dv

    return pl.pallas_call(
        body,
        grid=(_NBLK,),
        in_specs=[_FLAT, _FLAT_HI, _FLAT],
        out_specs=[_FLAT, _FLAT],
        out_shape=[
            jax.ShapeDtypeStruct((_NFR, 128), jnp.float32),
            jax.ShapeDtypeStruct((_NFR, 128), jnp.float32),
        ],
    )(cnt2, cnt2, x16)


def _tc_layer1(p0, g0, dinv, W128, b128):
    def body(pa_ref, pb_ref, g_ref, dv_ref, w_ref, b_ref, out_ref):
        dv = dv_ref[...]
        t = (pa_ref[...] + pb_ref[...] + g_ref[...]) * dv
        h = jnp.dot(t, w_ref[...], preferred_element_type=jnp.float32)
        out_ref[...] = jnp.maximum(h + b_ref[...], 0.0) * dv

    return pl.pallas_call(
        body,
        grid=(_NBLK,),
        in_specs=[_FLAT, _FLAT_HI, _FLAT, _FLAT, _W128, _B128],
        out_specs=_FLAT,
        out_shape=jax.ShapeDtypeStruct((_NFR, 128), jnp.float32),
    )(p0, p0, g0, dinv, W128, b128)


def _tc_layer2(p1, g1, dinv, Wa128, Wb128, ba128, bb128):
    def body(pa_ref, pb_ref, g_ref, dv_ref, wa_ref, wb_ref, ba_ref, bb_ref,
             ga_ref, gb_ref):
        dv = dv_ref[...]
        t = (pa_ref[...] + pb_ref[...] + g_ref[...]) * dv
        ha = jnp.dot(t, wa_ref[...], preferred_element_type=jnp.float32)
        hb = jnp.dot(t, wb_ref[...], preferred_element_type=jnp.float32)
        ga_ref[...] = jnp.maximum(ha + ba_ref[...], 0.0) * dv
        gb_ref[...] = jnp.maximum(hb + bb_ref[...], 0.0) * dv

    return pl.pallas_call(
        body,
        grid=(_NBLK,),
        in_specs=[_FLAT, _FLAT_HI, _FLAT, _FLAT, _W128, _W128, _B128,
                  _B128],
        out_specs=[_FLAT, _FLAT],
        out_shape=[
            jax.ShapeDtypeStruct((_NFR, 128), jnp.float32),
            jax.ShapeDtypeStruct((_NFR, 128), jnp.float32),
        ],
    )(p1, p1, g1, dinv, Wa128, Wb128, ba128, bb128)


def _tc_final(p2, ga, gb, dinv, lg, gn, W3, b3):
    def body(pa_ref, pb_ref, ga_ref, gb_ref, dv_ref, lg_ref, gn_ref,
             wa_ref, wb_ref, b_ref, out_ref):
        dv = dv_ref[...]
        ta = (pa_ref[...] + ga_ref[...]) * dv
        tb = (pb_ref[...] + gb_ref[...]) * dv
        z = (lg_ref[...] + gn_ref[...]) * _INV_TEMP
        z = z - jnp.max(z, axis=2, keepdims=True)
        e = jnp.exp(z)
        y = e / jnp.sum(e, axis=2, keepdims=True)
        r_io = lax.broadcasted_iota(jnp.int32, (_BF, 8, 1), 0)
        j_io = lax.broadcasted_iota(jnp.int32, (_BF, 8, 1), 1)
        n_id = (pl.program_id(0) * _BF + r_io) * 8 + j_io
        y = jnp.where(n_id < _N, y, 0.0)

        acc = jnp.zeros((_K, 64), jnp.float32)
        for j in range(8):
            taj = ta[:, j * _F:(j + 1) * _F]
            tbj = tb[:, j * _F:(j + 1) * _F]
            h = (jnp.dot(taj, wa_ref[...], preferred_element_type=jnp.float32)
                 + jnp.dot(tbj, wb_ref[...],
                           preferred_element_type=jnp.float32))
            h = jnp.maximum(h + b_ref[...], 0.0)
            acc = acc + lax.dot_general(
                y[:, j, :], h, (((0,), (0,)), ((), ())),
                preferred_element_type=jnp.float32)

        @pl.when(pl.program_id(0) == 0)
        def _():
            out_ref[...] = jnp.zeros_like(out_ref)

        out_ref[...] += acc

    lg8 = lg.reshape(_N // 8, 8, _K)
    gn8 = gn.reshape(_N // 8, 8, _K)
    return pl.pallas_call(
        body,
        grid=(_NBLK,),
        in_specs=[
            _FLAT, _FLAT_HI, _FLAT, _FLAT, _FLAT,
            pl.BlockSpec((_BF, 8, _K), lambda i: (i, 0, 0)),
            pl.BlockSpec((_BF, 8, _K), lambda i: (i, 0, 0)),
            pl.BlockSpec((_F, 64), lambda i: (0, 0)),
            pl.BlockSpec((_F, 64), lambda i: (0, 0)),
            pl.BlockSpec((1, 64), lambda i: (0, 0)),
        ],
        out_specs=pl.BlockSpec((_K, 64), lambda i: (0, 0)),
        out_shape=jax.ShapeDtypeStruct((_K, 64), jnp.float32),
    )(p2, p2, ga, gb, dinv, lg8, gn8, W3[:_F], W3[_F:], b3)


def kernel(x, edge_index, logits, gumbel_noise, W1, b1, W2, b2, W3, b3):
    src = edge_index[0]
    dst = edge_index[1]
    pad_e = _EPAD - _E
    src_p = jnp.concatenate([src, jnp.zeros((pad_e,), jnp.int32)]
                            ).reshape(_NCHUNK, _CH)
    dst_p = jnp.concatenate([dst, jnp.full((pad_e,), _N, jnp.int32)]
                            ).reshape(_NCHUNK, _CH)
    x16 = jnp.zeros((_NPAD, _F), jnp.float32).at[:_N, :2].set(x)
    z16 = jnp.zeros((_NPAD, _F), jnp.float32)
    ones = jnp.ones((_CH, _F), jnp.float32)

    eye8 = jnp.eye(8, dtype=jnp.float32)
    W1e = jnp.zeros((_F, _F), jnp.float32).at[:2].set(W1)
    W1k = jnp.kron(eye8, W1e)
    b1k = jnp.tile(b1, 8).reshape(1, 128)
    Wa = jnp.kron(eye8, W2[:, :16])
    Wb = jnp.kron(eye8, W2[:, 16:])
    ba = jnp.tile(b2[:16], 8).reshape(1, 128)
    bb = jnp.tile(b2[16:], 8).reshape(1, 128)

    def flat(a):
        return a.reshape(_NC * _NFR, 128)

    cnt = flat(_sc_degree(dst_p, ones, z16))
    dinv, g0 = _tc_prep(cnt, x16.reshape(_NFR, 128))
    g0t = g0.reshape(_NPAD, _F)
    p0 = flat(_sc_agg(src_p, dst_p, g0t, z16))
    g1 = _tc_layer1(p0, g0, dinv, W1k, b1k)
    p1 = flat(_sc_agg(src_p, dst_p, g1.reshape(_NPAD, _F), z16))
    ga, gb = _tc_layer2(p1, g1, dinv, Wa, Wb, ba, bb)
    p2 = flat(_sc_agg_split(src_p, dst_p, ga.reshape(_NPAD, _F),
                            gb.reshape(_NPAD, _F), z16))
    out = _tc_final(p2, ga, gb, dinv, logits, gumbel_noise, W3,
                    b3.reshape(1, 64))
    return out.reshape(1, -1)
